# Initial kernel scaffold; baseline (speedup 1.0000x reference)
import functools

import jax
import jax.numpy as jnp
from jax import lax
from jax.experimental import pallas as pl
from jax.experimental.pallas import tpu as pltpu
from jax.experimental.pallas import tpu_sc as plsc

B = 100
NPG = 100
D = 128
N = B * NPG
E = 320000
K1, K2, K3 = 50, 25, 13


_NC = 2
_NS = 16
_NW = _NC * _NS
_EPW = E // _NW
_BK = 80
_NB = _EPW // _BK
_RPS = N // _NS


def _sc_segsum_body(h_hbm, src_hbm, dst_hbm, zeros_hbm, out_hbm,
                    src_v, dst_v, rows_v, acc, sem):
    c = lax.axis_index("c")
    s = lax.axis_index("s")
    wid = s * _NC + c

    pltpu.sync_copy(zeros_hbm, acc.at[pl.ds(s * _RPS, _RPS)])
    plsc.subcore_barrier()

    def body(i, carry):
        base = wid * _EPW + i * _BK
        pltpu.sync_copy(src_hbm.at[pl.ds(base, _BK)], src_v)
        pltpu.sync_copy(dst_hbm.at[pl.ds(base, _BK)], dst_v)
        pltpu.async_copy(h_hbm.at[src_v], rows_v, sem).wait()
        pltpu.sync_copy(rows_v, acc.at[dst_v], add=True)
        return carry

    lax.fori_loop(0, _NB, body, 0)

    plsc.subcore_barrier()
    pltpu.sync_copy(acc.at[pl.ds(s * _RPS, _RPS)],
                    out_hbm.at[c, pl.ds(s * _RPS, _RPS)])


_sc_segsum = pl.kernel(
    _sc_segsum_body,
    out_type=jax.ShapeDtypeStruct((_NC, N, D), jnp.float32),
    mesh=plsc.VectorSubcoreMesh(core_axis_name="c", subcore_axis_name="s"),
    scratch_types=[
        pltpu.VMEM((_BK,), jnp.int32),
        pltpu.VMEM((_BK,), jnp.int32),
        pltpu.VMEM((_BK, D), jnp.float32),
        pltpu.VMEM_SHARED((N, D), jnp.float32),
        pltpu.SemaphoreType.DMA,
    ],
)


_RA = 1000


def _tca_body(part_ref, hp_ref, wrel_ref, brel_ref, wroot_ref, q_ref,
              h_ref, sc_ref):
    aggr = part_ref[0] + part_ref[1]
    t = (jnp.dot(aggr, wrel_ref[...], preferred_element_type=jnp.float32)
         + jnp.dot(hp_ref[...], wroot_ref[...],
                   preferred_element_type=jnp.float32)
         + brel_ref[...])
    h = jnp.maximum(t, 0.0)
    h_ref[...] = h
    u = jnp.sum(h * q_ref[...], axis=1, keepdims=True)
    sc_ref[...] = jnp.tanh(u)


def _tca(part, hprev, wrel, brel, wroot, q):
    return pl.pallas_call(
        _tca_body,
        grid=(N // _RA,),
        in_specs=[
            pl.BlockSpec((_NC, _RA, D), lambda i: (0, i, 0)),
            pl.BlockSpec((_RA, D), lambda i: (i, 0)),
            pl.BlockSpec((D, D), lambda i: (0, 0)),
            pl.BlockSpec((1, D), lambda i: (0, 0)),
            pl.BlockSpec((D, D), lambda i: (0, 0)),
            pl.BlockSpec((1, D), lambda i: (0, 0)),
        ],
        out_specs=[
            pl.BlockSpec((_RA, D), lambda i: (i, 0)),
            pl.BlockSpec((_RA, 1), lambda i: (i, 0)),
        ],
        out_shape=[
            jax.ShapeDtypeStruct((N, D), jnp.float32),
            jax.ShapeDtypeStruct((N, 1), jnp.float32),
        ],
    )(part, hprev, wrel, brel, wroot, q)


_NEG = jnp.float32(-3e38)


def _rank_keep(score, ordk, k):
    alive = ordk < NPG
    s = jnp.where(alive, score, _NEG)
    a = s[:, :, None]
    b = s[:, None, :]
    oi = ordk[:, :, None]
    oj = ordk[:, None, :]
    better = (b > a) | ((b == a) & (oj < oi))
    rank = jnp.sum(better.astype(jnp.float32), axis=2).astype(jnp.int32)
    keep = (rank < k) & alive
    return keep, rank


def _tcb_body(hg_ref, sg_ref, og_ref, hm_ref, on_ref, r_ref, *, k):
    score = sg_ref[...]
    ordk = og_ref[...]
    keep, rank = _rank_keep(score, ordk, k)
    on_ref[...] = jnp.where(keep, rank, NPG)
    m = jnp.where(keep, score, 0.0)
    hm = hg_ref[...] * m[:, :, None]
    hm_ref[...] = hm
    mx = jnp.max(jnp.where(keep[:, :, None], hm, _NEG), axis=1)
    mn = jnp.sum(hm, axis=1) * jnp.float32(1.0 / k)
    r_ref[...] = jnp.concatenate([mx, mn], axis=1)


def _tcb(hg, sg, og, k):
    return pl.pallas_call(
        functools.partial(_tcb_body, k=k),
        out_shape=[
            jax.ShapeDtypeStruct((B, NPG, D), jnp.float32),
            jax.ShapeDtypeStruct((B, NPG), jnp.int32),
            jax.ShapeDtypeStruct((B, 2 * D), jnp.float32),
        ],
    )(hg, sg, og)


def _tcc_body(hg_ref, sg_ref, og_ref, r1_ref, r2_ref,
              w1_ref, b1_ref, w2_ref, b2_ref, w3_ref, b3_ref, out_ref, *, k):
    score = sg_ref[...]
    keep, _ = _rank_keep(score, og_ref[...], k)
    m = jnp.where(keep, score, 0.0)
    hm = hg_ref[...] * m[:, :, None]
    mx = jnp.max(jnp.where(keep[:, :, None], hm, _NEG), axis=1)
    mn = jnp.sum(hm, axis=1) * jnp.float32(1.0 / k)
    z = r1_ref[...] + r2_ref[...] + jnp.concatenate([mx, mn], axis=1)
    z = jnp.maximum(jnp.dot(z, w1_ref[...],
                            preferred_element_type=jnp.float32)
                    + b1_ref[...], 0.0)
    z = jnp.maximum(jnp.dot(z, w2_ref[...],
                            preferred_element_type=jnp.float32)
                    + b2_ref[...], 0.0)
    logits = jnp.dot(z, w3_ref[...],
                     preferred_element_type=jnp.float32) + b3_ref[...]
    mxl = jnp.max(logits, axis=1, keepdims=True)
    lse = jnp.log(jnp.sum(jnp.exp(logits - mxl), axis=1, keepdims=True))
    out_ref[...] = logits - mxl - lse


def _tcc(hg, sg, og, r1, r2, w1, b1, w2, b2, w3, b3, k):
    return pl.pallas_call(
        functools.partial(_tcc_body, k=k),
        out_shape=jax.ShapeDtypeStruct((B, 10), jnp.float32),
    )(hg, sg, og, r1, r2, w1, b1, w2, b2, w3, b3)


def kernel(x, edge_index, batch, conv1_Wrel, conv1_brel, conv1_Wroot, p1,
           conv2_Wrel, conv2_brel, conv2_Wroot, p2,
           conv3_Wrel, conv3_brel, conv3_Wroot, p3,
           lin1_W, lin1_b, lin2_W, lin2_b, lin3_W, lin3_b):
    src = edge_index[0]
    dst = edge_index[1]
    zeros = jnp.zeros((_RPS, D), jnp.float32)
    ord0 = jnp.tile(jnp.arange(NPG, dtype=jnp.int32), B).reshape(B, NPG)

    def qv(p):
        return (p / (jnp.linalg.norm(p) + 1e-16)).reshape(1, D)

    def stage(hflat, wrel, brel, wroot, p):
        part = _sc_segsum(hflat, src, dst, zeros)
        h, sc = _tca(part, hflat, wrel, brel.reshape(1, D), wroot, qv(p))
        return h.reshape(B, NPG, D), sc.reshape(B, NPG)

    hg, sg = stage(x, conv1_Wrel, conv1_brel, conv1_Wroot, p1)
    hm, ord1, r1 = _tcb(hg, sg, ord0, K1)

    hg, sg = stage(hm.reshape(N, D), conv2_Wrel, conv2_brel, conv2_Wroot, p2)
    hm, ord2, r2 = _tcb(hg, sg, ord1, K2)

    hg, sg = stage(hm.reshape(N, D), conv3_Wrel, conv3_brel, conv3_Wroot, p3)
    return _tcc(hg, sg, ord2, r1, r2,
                lin1_W, lin1_b.reshape(1, D), lin2_W, lin2_b.reshape(1, 64),
                lin3_W, lin3_b.reshape(1, 10), K3)


# speedup vs baseline: 11.8301x; 11.8301x over previous
import functools

import jax
import jax.numpy as jnp
from jax import lax
from jax.experimental import pallas as pl
from jax.experimental.pallas import tpu as pltpu
from jax.experimental.pallas import tpu_sc as plsc

B = 100
NPG = 100
D = 128
N = B * NPG
E = 320000
K1, K2, K3 = 50, 25, 13


_NC = 2
_NS = 16
_NW = _NC * _NS
_EPW = E // _NW
_BK = 80
_NB = _EPW // _BK
_CH = 400
_NCH = N // _CH


def _sc_segsum_body(h_hbm, src_hbm, dst_hbm, zeros_hbm, out_hbm,
                    src_v, dst_v, rows_v, acc, sem):
    c = lax.axis_index("c")
    s = lax.axis_index("s")
    wid = s * _NC + c

    for j in range(2):
        cid = s + j * _NS

        @pl.when(cid < _NCH)
        def _():
            pltpu.sync_copy(zeros_hbm, acc.at[pl.ds(cid * _CH, _CH)])

    plsc.subcore_barrier()

    def body(i, carry):
        base = wid * _EPW + i * _BK
        pltpu.sync_copy(src_hbm.at[pl.ds(base, _BK)], src_v)
        pltpu.sync_copy(dst_hbm.at[pl.ds(base, _BK)], dst_v)
        pltpu.async_copy(h_hbm.at[src_v], rows_v, sem).wait()
        pltpu.sync_copy(rows_v, acc.at[dst_v], add=True)
        return carry

    lax.fori_loop(0, _NB, body, 0)

    plsc.subcore_barrier()
    for j in range(2):
        cid = s + j * _NS

        @pl.when(cid < _NCH)
        def _():
            pltpu.sync_copy(acc.at[pl.ds(cid * _CH, _CH)],
                            out_hbm.at[c, pl.ds(cid * _CH, _CH)])


@functools.cache
def _sc_segsum_fn():
    return pl.kernel(
        _sc_segsum_body,
        out_type=jax.ShapeDtypeStruct((_NC, N, D), jnp.float32),
        mesh=plsc.VectorSubcoreMesh(core_axis_name="c", subcore_axis_name="s"),
        scratch_types=[
            pltpu.VMEM((_BK,), jnp.int32),
            pltpu.VMEM((_BK,), jnp.int32),
            pltpu.VMEM((_BK, D), jnp.float32),
            pltpu.VMEM_SHARED((N, D), jnp.float32),
            pltpu.SemaphoreType.DMA,
        ],
    )


def _sc_segsum(h, src, dst, zeros):
    return _sc_segsum_fn()(h, src, dst, zeros)


_RA = 1000


def _tca_body(part_ref, hp_ref, wrel_ref, brel_ref, wroot_ref, q_ref,
              h_ref, sc_ref):
    aggr = part_ref[0] + part_ref[1]
    t = (jnp.dot(aggr, wrel_ref[...], preferred_element_type=jnp.float32)
         + brel_ref[...]
         + jnp.dot(hp_ref[...], wroot_ref[...],
                   preferred_element_type=jnp.float32))
    h = jnp.maximum(t, 0.0)
    h_ref[...] = h
    sc_ref[...] = jnp.dot(h, q_ref[...], preferred_element_type=jnp.float32)


def _tca(part, hprev, wrel, brel, wroot, q):
    return pl.pallas_call(
        _tca_body,
        grid=(N // _RA,),
        in_specs=[
            pl.BlockSpec((_NC, _RA, D), lambda i: (0, i, 0)),
            pl.BlockSpec((_RA, D), lambda i: (i, 0)),
            pl.BlockSpec((D, D), lambda i: (0, 0)),
            pl.BlockSpec((1, D), lambda i: (0, 0)),
            pl.BlockSpec((D, D), lambda i: (0, 0)),
            pl.BlockSpec((D, 1), lambda i: (0, 0)),
        ],
        out_specs=[
            pl.BlockSpec((_RA, D), lambda i: (i, 0)),
            pl.BlockSpec((_RA, 1), lambda i: (i, 0)),
        ],
        out_shape=[
            jax.ShapeDtypeStruct((N, D), jnp.float32),
            jax.ShapeDtypeStruct((N, 1), jnp.float32),
        ],
    )(part, hprev, wrel, brel, wroot, q)


_NEG = -3e38


def _rank_keep(score, ordk, k):
    alive = ordk < NPG
    s = jnp.where(alive, score, _NEG)
    a = s[:, :, None]
    b = s[:, None, :]
    oi = ordk[:, :, None]
    oj = ordk[:, None, :]
    better = (b > a) | ((b == a) & (oj < oi))
    rank = jnp.sum(better.astype(jnp.float32), axis=2).astype(jnp.int32)
    keep = (rank < k) & alive
    return keep, rank


def _tcb_body(pn_ref, hg_ref, sg_ref, og_ref, hm_ref, on_ref, r_ref, *, k):
    score = jnp.tanh(sg_ref[...] / pn_ref[0, 0])
    ordk = og_ref[...]
    keep, rank = _rank_keep(score, ordk, k)
    on_ref[...] = jnp.where(keep, rank, NPG)
    m = jnp.where(keep, score, 0.0)
    hm = hg_ref[...] * m[:, :, None]
    hm_ref[...] = hm
    mx = jnp.max(jnp.where(keep[:, :, None], hm, _NEG), axis=1)
    mn = jnp.sum(hm, axis=1) / k
    r_ref[...] = jnp.concatenate([mx, mn], axis=1)


def _tcb(pn, hg, sg, og, k):
    return pl.pallas_call(
        functools.partial(_tcb_body, k=k),
        in_specs=[
            pl.BlockSpec(memory_space=pltpu.SMEM),
            pl.BlockSpec(memory_space=pltpu.VMEM),
            pl.BlockSpec(memory_space=pltpu.VMEM),
            pl.BlockSpec(memory_space=pltpu.VMEM),
        ],
        out_shape=[
            jax.ShapeDtypeStruct((B, NPG, D), jnp.float32),
            jax.ShapeDtypeStruct((B, NPG), jnp.int32),
            jax.ShapeDtypeStruct((B, 2 * D), jnp.float32),
        ],
    )(pn, hg, sg, og)


def _tcc_body(pn_ref, hg_ref, sg_ref, og_ref, r1_ref, r2_ref,
              w1_ref, b1_ref, w2_ref, b2_ref, w3_ref, b3_ref, out_ref, *, k):
    score = jnp.tanh(sg_ref[...] / pn_ref[0, 0])
    keep, _ = _rank_keep(score, og_ref[...], k)
    m = jnp.where(keep, score, 0.0)
    hm = hg_ref[...] * m[:, :, None]
    mx = jnp.max(jnp.where(keep[:, :, None], hm, _NEG), axis=1)
    mn = jnp.sum(hm, axis=1) / k
    z = r1_ref[...] + r2_ref[...] + jnp.concatenate([mx, mn], axis=1)
    z = jnp.maximum(jnp.dot(z, w1_ref[...],
                            preferred_element_type=jnp.float32)
                    + b1_ref[...], 0.0)
    z = jnp.maximum(jnp.dot(z, w2_ref[...],
                            preferred_element_type=jnp.float32)
                    + b2_ref[...], 0.0)
    logits = jnp.dot(z, w3_ref[...],
                     preferred_element_type=jnp.float32) + b3_ref[...]
    mxl = jnp.max(logits, axis=1, keepdims=True)
    lse = jnp.log(jnp.sum(jnp.exp(logits - mxl), axis=1, keepdims=True))
    out_ref[...] = logits - mxl - lse


def _tcc(pn, hg, sg, og, r1, r2, w1, b1, w2, b2, w3, b3, k):
    return pl.pallas_call(
        functools.partial(_tcc_body, k=k),
        in_specs=[pl.BlockSpec(memory_space=pltpu.SMEM)]
        + [pl.BlockSpec(memory_space=pltpu.VMEM)] * 11,
        out_shape=jax.ShapeDtypeStruct((B, 10), jnp.float32),
    )(pn, hg, sg, og, r1, r2, w1, b1, w2, b2, w3, b3)


def kernel(x, edge_index, batch, conv1_Wrel, conv1_brel, conv1_Wroot, p1,
           conv2_Wrel, conv2_brel, conv2_Wroot, p2,
           conv3_Wrel, conv3_brel, conv3_Wroot, p3,
           lin1_W, lin1_b, lin2_W, lin2_b, lin3_W, lin3_b):
    src = edge_index[0]
    dst = edge_index[1]
    zeros = jnp.zeros((_CH, D), jnp.float32)
    ord0 = jnp.tile(jnp.arange(NPG, dtype=jnp.int32), B).reshape(B, NPG)

    def pnorm(p):
        return (jnp.linalg.norm(p) + 1e-16).reshape(1, 1)

    def stage(hflat, wrel, brel, wroot, p):
        part = _sc_segsum(hflat, src, dst, zeros)
        h, u = _tca(part, hflat, wrel, brel.reshape(1, D), wroot,
                    p.reshape(D, 1))
        return h.reshape(B, NPG, D), u.reshape(B, NPG)

    hg, ug = stage(x, conv1_Wrel, conv1_brel, conv1_Wroot, p1)
    hm, ord1, r1 = _tcb(pnorm(p1), hg, ug, ord0, K1)

    hg, ug = stage(hm.reshape(N, D), conv2_Wrel, conv2_brel, conv2_Wroot, p2)
    hm, ord2, r2 = _tcb(pnorm(p2), hg, ug, ord1, K2)

    hg, ug = stage(hm.reshape(N, D), conv3_Wrel, conv3_brel, conv3_Wroot, p3)
    return _tcc(pnorm(p3), hg, ug, ord2, r1, r2,
                lin1_W, lin1_b.reshape(1, D), lin2_W, lin2_b.reshape(1, 64),
                lin3_W, lin3_b.reshape(1, 10), K3)


# speedup vs baseline: 23.8681x; 2.0176x over previous
import functools

import jax
import jax.numpy as jnp
from jax import lax
from jax.experimental import pallas as pl
from jax.experimental.pallas import tpu as pltpu
from jax.experimental.pallas import tpu_sc as plsc

B = 100
NPG = 100
D = 128
N = B * NPG
E = 320000
K1, K2, K3 = 50, 25, 13


_NC = 2
_NS = 16
_NW = _NC * _NS
_EPW = E // _NW
_BK = 128
_NB = _EPW // _BK
_TL = _EPW - _NB * _BK
_CH = 400
_NCH = N // _CH


def _sc_segsum_body(h_hbm, src_hbm, dst_hbm, zeros_hbm, out_hbm,
                    src_all, d0, d1, rows0, rows1, dt, rowst, acc,
                    gsem0, gsem1, tsem):
    c = lax.axis_index("c")
    s = lax.axis_index("s")
    wid = s * _NC + c
    base = wid * _EPW

    for j in range(2):
        cid = s + j * _NS

        @pl.when(cid < _NCH)
        def _():
            pltpu.sync_copy(zeros_hbm, acc.at[pl.ds(cid * _CH, _CH)])

    pltpu.sync_copy(src_hbm.at[pl.ds(base, _EPW)], src_all)
    plsc.subcore_barrier()

    pltpu.async_copy(h_hbm.at[src_all.at[pl.ds(0, _BK)]], rows0, gsem0)
    pltpu.async_copy(h_hbm.at[src_all.at[pl.ds(_BK, _BK)]], rows1, gsem1)

    def slot(bid, dv, rows, gsem):
        pltpu.make_async_copy(h_hbm.at[src_all.at[pl.ds(0, _BK)]],
                              rows, gsem).wait()
        pltpu.sync_copy(dst_hbm.at[pl.ds(base + bid * _BK, _BK)], dv)
        pltpu.sync_copy(rows, acc.at[dv], add=True)

        @pl.when(bid + 2 < _NB)
        def _():
            nxt = (bid + 2) * _BK
            pltpu.async_copy(h_hbm.at[src_all.at[pl.ds(nxt, _BK)]],
                             rows, gsem)

    def body(jj, carry):
        slot(2 * jj, d0, rows0, gsem0)
        slot(2 * jj + 1, d1, rows1, gsem1)
        return carry

    lax.fori_loop(0, _NB // 2, body, 0)

    pltpu.async_copy(h_hbm.at[src_all.at[pl.ds(_NB * _BK, _TL)]],
                     rowst, tsem).wait()
    pltpu.sync_copy(dst_hbm.at[pl.ds(base + _NB * _BK, _TL)], dt)
    pltpu.sync_copy(rowst, acc.at[dt], add=True)

    plsc.subcore_barrier()
    for j in range(2):
        cid = s + j * _NS

        @pl.when(cid < _NCH)
        def _():
            pltpu.sync_copy(acc.at[pl.ds(cid * _CH, _CH)],
                            out_hbm.at[c, pl.ds(cid * _CH, _CH)])


@functools.cache
def _sc_segsum_fn():
    return pl.kernel(
        _sc_segsum_body,
        out_type=jax.ShapeDtypeStruct((_NC, N, D), jnp.float32),
        mesh=plsc.VectorSubcoreMesh(core_axis_name="c", subcore_axis_name="s"),
        scratch_types=[
            pltpu.VMEM((_EPW,), jnp.int32),
            pltpu.VMEM((_BK,), jnp.int32),
            pltpu.VMEM((_BK,), jnp.int32),
            pltpu.VMEM((_BK, D), jnp.float32),
            pltpu.VMEM((_BK, D), jnp.float32),
            pltpu.VMEM((_TL,), jnp.int32),
            pltpu.VMEM((_TL, D), jnp.float32),
            pltpu.VMEM_SHARED((N, D), jnp.float32),
            pltpu.SemaphoreType.DMA,
            pltpu.SemaphoreType.DMA,
            pltpu.SemaphoreType.DMA,
        ],
    )


def _sc_segsum(h, src, dst, zeros):
    return _sc_segsum_fn()(h, src, dst, zeros)


_RA = 1000


def _tca_body(part_ref, hp_ref, wrel_ref, brel_ref, wroot_ref, q_ref,
              h_ref, sc_ref):
    aggr = part_ref[0] + part_ref[1]
    t = (jnp.dot(aggr, wrel_ref[...], preferred_element_type=jnp.float32)
         + brel_ref[...]
         + jnp.dot(hp_ref[...], wroot_ref[...],
                   preferred_element_type=jnp.float32))
    h = jnp.maximum(t, 0.0)
    h_ref[...] = h
    sc_ref[...] = jnp.dot(h, q_ref[...], preferred_element_type=jnp.float32)


def _tca(part, hprev, wrel, brel, wroot, q):
    return pl.pallas_call(
        _tca_body,
        grid=(N // _RA,),
        in_specs=[
            pl.BlockSpec((_NC, _RA, D), lambda i: (0, i, 0)),
            pl.BlockSpec((_RA, D), lambda i: (i, 0)),
            pl.BlockSpec((D, D), lambda i: (0, 0)),
            pl.BlockSpec((1, D), lambda i: (0, 0)),
            pl.BlockSpec((D, D), lambda i: (0, 0)),
            pl.BlockSpec((D, 1), lambda i: (0, 0)),
        ],
        out_specs=[
            pl.BlockSpec((_RA, D), lambda i: (i, 0)),
            pl.BlockSpec((_RA, 1), lambda i: (i, 0)),
        ],
        out_shape=[
            jax.ShapeDtypeStruct((N, D), jnp.float32),
            jax.ShapeDtypeStruct((N, 1), jnp.float32),
        ],
    )(part, hprev, wrel, brel, wroot, q)


_NEG = -3e38


def _rank_keep(score, ordk, k):
    alive = ordk < NPG
    s = jnp.where(alive, score, _NEG)
    a = s[:, :, None]
    b = s[:, None, :]
    oi = ordk[:, :, None]
    oj = ordk[:, None, :]
    better = (b > a) | ((b == a) & (oj < oi))
    rank = jnp.sum(better.astype(jnp.float32), axis=2).astype(jnp.int32)
    keep = (rank < k) & alive
    return keep, rank


def _tcb_body(pn_ref, hg_ref, sg_ref, og_ref, hm_ref, on_ref, r_ref, *, k):
    score = jnp.tanh(sg_ref[...] / pn_ref[0, 0])
    ordk = og_ref[...]
    keep, rank = _rank_keep(score, ordk, k)
    on_ref[...] = jnp.where(keep, rank, NPG)
    m = jnp.where(keep, score, 0.0)
    hm = hg_ref[...] * m[:, :, None]
    hm_ref[...] = hm
    mx = jnp.max(jnp.where(keep[:, :, None], hm, _NEG), axis=1)
    mn = jnp.sum(hm, axis=1) / k
    r_ref[...] = jnp.concatenate([mx, mn], axis=1)


def _tcb(pn, hg, sg, og, k):
    return pl.pallas_call(
        functools.partial(_tcb_body, k=k),
        in_specs=[
            pl.BlockSpec(memory_space=pltpu.SMEM),
            pl.BlockSpec(memory_space=pltpu.VMEM),
            pl.BlockSpec(memory_space=pltpu.VMEM),
            pl.BlockSpec(memory_space=pltpu.VMEM),
        ],
        out_shape=[
            jax.ShapeDtypeStruct((B, NPG, D), jnp.float32),
            jax.ShapeDtypeStruct((B, NPG), jnp.int32),
            jax.ShapeDtypeStruct((B, 2 * D), jnp.float32),
        ],
    )(pn, hg, sg, og)


def _tcc_body(pn_ref, hg_ref, sg_ref, og_ref, r1_ref, r2_ref,
              w1_ref, b1_ref, w2_ref, b2_ref, w3_ref, b3_ref, out_ref, *, k):
    score = jnp.tanh(sg_ref[...] / pn_ref[0, 0])
    keep, _ = _rank_keep(score, og_ref[...], k)
    m = jnp.where(keep, score, 0.0)
    hm = hg_ref[...] * m[:, :, None]
    mx = jnp.max(jnp.where(keep[:, :, None], hm, _NEG), axis=1)
    mn = jnp.sum(hm, axis=1) / k
    z = r1_ref[...] + r2_ref[...] + jnp.concatenate([mx, mn], axis=1)
    z = jnp.maximum(jnp.dot(z, w1_ref[...],
                            preferred_element_type=jnp.float32)
                    + b1_ref[...], 0.0)
    z = jnp.maximum(jnp.dot(z, w2_ref[...],
                            preferred_element_type=jnp.float32)
                    + b2_ref[...], 0.0)
    logits = jnp.dot(z, w3_ref[...],
                     preferred_element_type=jnp.float32) + b3_ref[...]
    mxl = jnp.max(logits, axis=1, keepdims=True)
    lse = jnp.log(jnp.sum(jnp.exp(logits - mxl), axis=1, keepdims=True))
    out_ref[...] = logits - mxl - lse


def _tcc(pn, hg, sg, og, r1, r2, w1, b1, w2, b2, w3, b3, k):
    return pl.pallas_call(
        functools.partial(_tcc_body, k=k),
        in_specs=[pl.BlockSpec(memory_space=pltpu.SMEM)]
        + [pl.BlockSpec(memory_space=pltpu.VMEM)] * 11,
        out_shape=jax.ShapeDtypeStruct((B, 10), jnp.float32),
    )(pn, hg, sg, og, r1, r2, w1, b1, w2, b2, w3, b3)


def kernel(x, edge_index, batch, conv1_Wrel, conv1_brel, conv1_Wroot, p1,
           conv2_Wrel, conv2_brel, conv2_Wroot, p2,
           conv3_Wrel, conv3_brel, conv3_Wroot, p3,
           lin1_W, lin1_b, lin2_W, lin2_b, lin3_W, lin3_b):
    src = edge_index[0]
    dst = edge_index[1]
    zeros = jnp.zeros((_CH, D), jnp.float32)
    ord0 = jnp.tile(jnp.arange(NPG, dtype=jnp.int32), B).reshape(B, NPG)

    def pnorm(p):
        return (jnp.linalg.norm(p) + 1e-16).reshape(1, 1)

    def stage(hflat, wrel, brel, wroot, p):
        part = _sc_segsum(hflat, src, dst, zeros)
        h, u = _tca(part, hflat, wrel, brel.reshape(1, D), wroot,
                    p.reshape(D, 1))
        return h.reshape(B, NPG, D), u.reshape(B, NPG)

    hg, ug = stage(x, conv1_Wrel, conv1_brel, conv1_Wroot, p1)
    hm, ord1, r1 = _tcb(pnorm(p1), hg, ug, ord0, K1)

    hg, ug = stage(hm.reshape(N, D), conv2_Wrel, conv2_brel, conv2_Wroot, p2)
    hm, ord2, r2 = _tcb(pnorm(p2), hg, ug, ord1, K2)

    hg, ug = stage(hm.reshape(N, D), conv3_Wrel, conv3_brel, conv3_Wroot, p3)
    return _tcc(pnorm(p3), hg, ug, ord2, r1, r2,
                lin1_W, lin1_b.reshape(1, D), lin2_W, lin2_b.reshape(1, 64),
                lin3_W, lin3_b.reshape(1, 10), K3)


# speedup vs baseline: 26.2713x; 1.1007x over previous
import functools

import jax
import jax.numpy as jnp
from jax import lax
from jax.experimental import pallas as pl
from jax.experimental.pallas import tpu as pltpu
from jax.experimental.pallas import tpu_sc as plsc

B = 100
NPG = 100
D = 128
N = B * NPG
E = 320000
K1, K2, K3 = 50, 25, 13


_NC = 2
_NS = 16
_NW = _NC * _NS
_EPW = E // _NW
_BK = 128
_NB = _EPW // _BK
_TL = _EPW - _NB * _BK
_CH = 400
_NCH = N // _CH


def _sc_segsum_body(h_hbm, src_hbm, dst_hbm, zeros_hbm, out_hbm,
                    src_all, d0, d1, rows0, rows1, dt, rowst, acc,
                    gsem0, gsem1, dsem0, dsem1, tsem):
    c = lax.axis_index("c")
    s = lax.axis_index("s")
    wid = s * _NC + c
    base = wid * _EPW

    for j in range(2):
        cid = s + j * _NS

        @pl.when(cid < _NCH)
        def _():
            pltpu.sync_copy(zeros_hbm, acc.at[pl.ds(cid * _CH, _CH)])

    pltpu.sync_copy(src_hbm.at[pl.ds(base, _EPW)], src_all)
    plsc.subcore_barrier()

    pltpu.async_copy(h_hbm.at[src_all.at[pl.ds(0, _BK)]], rows0, gsem0)
    pltpu.async_copy(h_hbm.at[src_all.at[pl.ds(_BK, _BK)]], rows1, gsem1)
    pltpu.async_copy(dst_hbm.at[pl.ds(base, _BK)], d0, dsem0)
    pltpu.async_copy(dst_hbm.at[pl.ds(base + _BK, _BK)], d1, dsem1)

    def slot(bid, dv, rows, gsem, dsem):
        pltpu.make_async_copy(dst_hbm.at[pl.ds(base, _BK)], dv, dsem).wait()
        pltpu.make_async_copy(h_hbm.at[src_all.at[pl.ds(0, _BK)]],
                              rows, gsem).wait()
        pltpu.sync_copy(rows, acc.at[dv], add=True)

        @pl.when(bid + 2 < _NB)
        def _():
            nxt = (bid + 2) * _BK
            pltpu.async_copy(h_hbm.at[src_all.at[pl.ds(nxt, _BK)]],
                             rows, gsem)
            pltpu.async_copy(dst_hbm.at[pl.ds(base + nxt, _BK)], dv, dsem)

    def body(jj, carry):
        slot(2 * jj, d0, rows0, gsem0, dsem0)
        slot(2 * jj + 1, d1, rows1, gsem1, dsem1)
        return carry

    lax.fori_loop(0, _NB // 2, body, 0)

    pltpu.async_copy(h_hbm.at[src_all.at[pl.ds(_NB * _BK, _TL)]],
                     rowst, tsem).wait()
    pltpu.sync_copy(dst_hbm.at[pl.ds(base + _NB * _BK, _TL)], dt)
    pltpu.sync_copy(rowst, acc.at[dt], add=True)

    plsc.subcore_barrier()
    for j in range(2):
        cid = s + j * _NS

        @pl.when(cid < _NCH)
        def _():
            pltpu.sync_copy(acc.at[pl.ds(cid * _CH, _CH)],
                            out_hbm.at[c, pl.ds(cid * _CH, _CH)])


@functools.cache
def _sc_segsum_fn():
    return pl.kernel(
        _sc_segsum_body,
        out_type=jax.ShapeDtypeStruct((_NC, N, D), jnp.float32),
        mesh=plsc.VectorSubcoreMesh(core_axis_name="c", subcore_axis_name="s"),
        scratch_types=[
            pltpu.VMEM((_EPW,), jnp.int32),
            pltpu.VMEM((_BK,), jnp.int32),
            pltpu.VMEM((_BK,), jnp.int32),
            pltpu.VMEM((_BK, D), jnp.float32),
            pltpu.VMEM((_BK, D), jnp.float32),
            pltpu.VMEM((_TL,), jnp.int32),
            pltpu.VMEM((_TL, D), jnp.float32),
            pltpu.VMEM_SHARED((N, D), jnp.float32),
            pltpu.SemaphoreType.DMA,
            pltpu.SemaphoreType.DMA,
            pltpu.SemaphoreType.DMA,
            pltpu.SemaphoreType.DMA,
            pltpu.SemaphoreType.DMA,
        ],
    )


def _sc_segsum(h, src, dst, zeros):
    return _sc_segsum_fn()(h, src, dst, zeros)


_RA = 1000


def _tca_body(part_ref, hp_ref, wrel_ref, brel_ref, wroot_ref, q_ref,
              h_ref, sc_ref):
    aggr = part_ref[0] + part_ref[1]
    t = (jnp.dot(aggr, wrel_ref[...], preferred_element_type=jnp.float32)
         + brel_ref[...]
         + jnp.dot(hp_ref[...], wroot_ref[...],
                   preferred_element_type=jnp.float32))
    h = jnp.maximum(t, 0.0)
    h_ref[...] = h
    sc_ref[...] = jnp.dot(h, q_ref[...], preferred_element_type=jnp.float32)


def _tca(part, hprev, wrel, brel, wroot, q):
    return pl.pallas_call(
        _tca_body,
        grid=(N // _RA,),
        in_specs=[
            pl.BlockSpec((_NC, _RA, D), lambda i: (0, i, 0)),
            pl.BlockSpec((_RA, D), lambda i: (i, 0)),
            pl.BlockSpec((D, D), lambda i: (0, 0)),
            pl.BlockSpec((1, D), lambda i: (0, 0)),
            pl.BlockSpec((D, D), lambda i: (0, 0)),
            pl.BlockSpec((D, 1), lambda i: (0, 0)),
        ],
        out_specs=[
            pl.BlockSpec((_RA, D), lambda i: (i, 0)),
            pl.BlockSpec((_RA, 1), lambda i: (i, 0)),
        ],
        out_shape=[
            jax.ShapeDtypeStruct((N, D), jnp.float32),
            jax.ShapeDtypeStruct((N, 1), jnp.float32),
        ],
    )(part, hprev, wrel, brel, wroot, q)


_NEG = -3e38


def _rank_keep(score, ordk, k):
    alive = ordk < NPG
    s = jnp.where(alive, score, _NEG)
    a = s[:, :, None]
    b = s[:, None, :]
    oi = ordk[:, :, None]
    oj = ordk[:, None, :]
    better = (b > a) | ((b == a) & (oj < oi))
    rank = jnp.sum(better.astype(jnp.float32), axis=2).astype(jnp.int32)
    keep = (rank < k) & alive
    return keep, rank


def _tcb_body(pn_ref, hg_ref, sg_ref, og_ref, hm_ref, on_ref, r_ref, *, k):
    score = jnp.tanh(sg_ref[...] / pn_ref[0, 0])
    ordk = og_ref[...]
    keep, rank = _rank_keep(score, ordk, k)
    on_ref[...] = jnp.where(keep, rank, NPG)
    m = jnp.where(keep, score, 0.0)
    hm = hg_ref[...] * m[:, :, None]
    hm_ref[...] = hm
    mx = jnp.max(jnp.where(keep[:, :, None], hm, _NEG), axis=1)
    mn = jnp.sum(hm, axis=1) / k
    r_ref[...] = jnp.concatenate([mx, mn], axis=1)


def _tcb(pn, hg, sg, og, k):
    return pl.pallas_call(
        functools.partial(_tcb_body, k=k),
        in_specs=[
            pl.BlockSpec(memory_space=pltpu.SMEM),
            pl.BlockSpec(memory_space=pltpu.VMEM),
            pl.BlockSpec(memory_space=pltpu.VMEM),
            pl.BlockSpec(memory_space=pltpu.VMEM),
        ],
        out_shape=[
            jax.ShapeDtypeStruct((B, NPG, D), jnp.float32),
            jax.ShapeDtypeStruct((B, NPG), jnp.int32),
            jax.ShapeDtypeStruct((B, 2 * D), jnp.float32),
        ],
    )(pn, hg, sg, og)


def _tcc_body(pn_ref, hg_ref, sg_ref, og_ref, r1_ref, r2_ref,
              w1_ref, b1_ref, w2_ref, b2_ref, w3_ref, b3_ref, out_ref, *, k):
    score = jnp.tanh(sg_ref[...] / pn_ref[0, 0])
    keep, _ = _rank_keep(score, og_ref[...], k)
    m = jnp.where(keep, score, 0.0)
    hm = hg_ref[...] * m[:, :, None]
    mx = jnp.max(jnp.where(keep[:, :, None], hm, _NEG), axis=1)
    mn = jnp.sum(hm, axis=1) / k
    z = r1_ref[...] + r2_ref[...] + jnp.concatenate([mx, mn], axis=1)
    z = jnp.maximum(jnp.dot(z, w1_ref[...],
                            preferred_element_type=jnp.float32)
                    + b1_ref[...], 0.0)
    z = jnp.maximum(jnp.dot(z, w2_ref[...],
                            preferred_element_type=jnp.float32)
                    + b2_ref[...], 0.0)
    logits = jnp.dot(z, w3_ref[...],
                     preferred_element_type=jnp.float32) + b3_ref[...]
    mxl = jnp.max(logits, axis=1, keepdims=True)
    lse = jnp.log(jnp.sum(jnp.exp(logits - mxl), axis=1, keepdims=True))
    out_ref[...] = logits - mxl - lse


def _tcc(pn, hg, sg, og, r1, r2, w1, b1, w2, b2, w3, b3, k):
    return pl.pallas_call(
        functools.partial(_tcc_body, k=k),
        in_specs=[pl.BlockSpec(memory_space=pltpu.SMEM)]
        + [pl.BlockSpec(memory_space=pltpu.VMEM)] * 11,
        out_shape=jax.ShapeDtypeStruct((B, 10), jnp.float32),
    )(pn, hg, sg, og, r1, r2, w1, b1, w2, b2, w3, b3)


def kernel(x, edge_index, batch, conv1_Wrel, conv1_brel, conv1_Wroot, p1,
           conv2_Wrel, conv2_brel, conv2_Wroot, p2,
           conv3_Wrel, conv3_brel, conv3_Wroot, p3,
           lin1_W, lin1_b, lin2_W, lin2_b, lin3_W, lin3_b):
    src = edge_index[0]
    dst = edge_index[1]
    zeros = jnp.zeros((_CH, D), jnp.float32)
    ord0 = jnp.tile(jnp.arange(NPG, dtype=jnp.int32), B).reshape(B, NPG)

    def pnorm(p):
        return (jnp.linalg.norm(p) + 1e-16).reshape(1, 1)

    def stage(hflat, wrel, brel, wroot, p):
        part = _sc_segsum(hflat, src, dst, zeros)
        h, u = _tca(part, hflat, wrel, brel.reshape(1, D), wroot,
                    p.reshape(D, 1))
        return h.reshape(B, NPG, D), u.reshape(B, NPG)

    hg, ug = stage(x, conv1_Wrel, conv1_brel, conv1_Wroot, p1)
    hm, ord1, r1 = _tcb(pnorm(p1), hg, ug, ord0, K1)

    hg, ug = stage(hm.reshape(N, D), conv2_Wrel, conv2_brel, conv2_Wroot, p2)
    hm, ord2, r2 = _tcb(pnorm(p2), hg, ug, ord1, K2)

    hg, ug = stage(hm.reshape(N, D), conv3_Wrel, conv3_brel, conv3_Wroot, p3)
    return _tcc(pnorm(p3), hg, ug, ord2, r1, r2,
                lin1_W, lin1_b.reshape(1, D), lin2_W, lin2_b.reshape(1, 64),
                lin3_W, lin3_b.reshape(1, 10), K3)
